# baseline (speedup 1.0000x reference)
import jax
import jax.numpy as jnp
from jax import lax
from jax.experimental import pallas as pl
from jax.experimental.pallas import tpu as pltpu
from jax.experimental.pallas import tpu_sc as plsc

_NUM_EMBEDDINGS = 100000
_DIM = 128
_BATCH = 4096
_HIST = 50

_INFO = plsc.get_sparse_core_info()
_NC = _INFO.num_cores
_NS = _INFO.num_subcores
_NW = _NC * _NS

_K = 2
_EPC = _BATCH // _K
_EPW = _EPC // _NW
_NBUF = 4
_LOOKAHEAD = 2


def _gather_body(idx_hbm, table_hbm, out_hbm, idx_v, rows_v, gsems, ssems):
    wid = lax.axis_index("s") * _NC + lax.axis_index("c")
    base = wid * _EPW

    pltpu.sync_copy(idx_hbm.at[wid], idx_v)

    def start_gather(c, b):
        pltpu.async_copy(table_hbm.at[idx_v.at[c]], rows_v.at[b], gsems.at[b])

    def wait_gather(b):
        pltpu.make_async_copy(
            table_hbm.at[idx_v.at[0]], rows_v.at[b], gsems.at[b]
        ).wait()

    def start_store(c, b):
        pltpu.async_copy(rows_v.at[b], out_hbm.at[base + c], ssems.at[b])

    def wait_store(b):
        pltpu.make_async_copy(
            rows_v.at[b], out_hbm.at[base], ssems.at[b]
        ).wait()

    for c0 in range(_LOOKAHEAD):
        start_gather(c0, c0)

    def group_body(g, carry):
        del carry
        for b in range(_NBUF):
            c = g * _NBUF + b
            tb = (b + _LOOKAHEAD) % _NBUF

            @pl.when(c + _LOOKAHEAD < _EPW)
            def _():
                @pl.when(c + _LOOKAHEAD >= _NBUF)
                def _():
                    wait_store(tb)

                start_gather(c + _LOOKAHEAD, tb)

            wait_gather(b)
            start_store(c, b)
        return 0

    lax.fori_loop(0, _EPW // _NBUF, group_body, 0)

    for b in range(_NBUF):
        wait_store(b)


@jax.jit
def kernel(x, table):
    idx = x.reshape(_K, _NW, _EPW, _HIST).astype(jnp.int32)
    call = pl.kernel(
        _gather_body,
        out_type=jax.ShapeDtypeStruct((_EPC, _HIST, _DIM), jnp.float32),
        mesh=plsc.VectorSubcoreMesh(core_axis_name="c", subcore_axis_name="s"),
        scratch_types=[
            pltpu.VMEM((_EPW, _HIST), jnp.int32),
            pltpu.VMEM((_NBUF, _HIST, _DIM), jnp.float32),
            pltpu.SemaphoreType.DMA((_NBUF,)),
            pltpu.SemaphoreType.DMA((_NBUF,)),
        ],
    )
    out = jnp.zeros((_BATCH, _HIST, _DIM), jnp.float32)
    for k in range(_K):
        part = call(idx[k], table)
        out = lax.dynamic_update_slice(out, part, (k * _EPC, 0, 0))
    return out


# speedup vs baseline: 1.6659x; 1.6659x over previous
import jax
import jax.numpy as jnp
from jax import lax
from jax.experimental import pallas as pl
from jax.experimental.pallas import tpu as pltpu
from jax.experimental.pallas import tpu_sc as plsc

_NUM_EMBEDDINGS = 100000
_DIM = 128
_BATCH = 4096
_HIST = 50

_INFO = plsc.get_sparse_core_info()
_NC = _INFO.num_cores
_NS = _INFO.num_subcores
_NW = _NC * _NS

_EPW = _BATCH // _NW
_NBUF = 4
_LOOKAHEAD = 2


def _gather_body(idx_hbm, table_hbm, out_hbm, idx_v, rows_v, gsems, ssems):
    wid = lax.axis_index("s") * _NC + lax.axis_index("c")
    base = wid * _EPW

    pltpu.sync_copy(idx_hbm.at[wid], idx_v)

    def start_gather(c, b):
        pltpu.async_copy(table_hbm.at[idx_v.at[c]], rows_v.at[b], gsems.at[b])

    def wait_gather(b):
        pltpu.make_async_copy(
            table_hbm.at[idx_v.at[0]], rows_v.at[b], gsems.at[b]
        ).wait()

    def start_store(c, b):
        pltpu.async_copy(rows_v.at[b], out_hbm.at[base + c], ssems.at[b])

    def wait_store(b):
        pltpu.make_async_copy(
            rows_v.at[b], out_hbm.at[base], ssems.at[b]
        ).wait()

    for c0 in range(_LOOKAHEAD):
        start_gather(c0, c0)

    def group_body(g, carry):
        del carry
        for b in range(_NBUF):
            c = g * _NBUF + b
            tb = (b + _LOOKAHEAD) % _NBUF

            @pl.when(c + _LOOKAHEAD < _EPW)
            def _():
                @pl.when(c + _LOOKAHEAD >= _NBUF)
                def _():
                    wait_store(tb)

                start_gather(c + _LOOKAHEAD, tb)

            wait_gather(b)
            start_store(c, b)
        return 0

    lax.fori_loop(0, _EPW // _NBUF, group_body, 0)

    for b in range(_NBUF):
        wait_store(b)


@jax.jit
def kernel(x, table):
    idx = x.reshape(_NW, _EPW, _HIST).astype(jnp.int32)
    call = pl.kernel(
        _gather_body,
        out_type=jax.ShapeDtypeStruct((_BATCH, _HIST, _DIM), jnp.float32),
        mesh=plsc.VectorSubcoreMesh(core_axis_name="c", subcore_axis_name="s"),
        scratch_types=[
            pltpu.VMEM((_EPW, _HIST), jnp.int32),
            pltpu.VMEM((_NBUF, _HIST, _DIM), jnp.float32),
            pltpu.SemaphoreType.DMA((_NBUF,)),
            pltpu.SemaphoreType.DMA((_NBUF,)),
        ],
    )
    return call(idx, table)
